# Initial kernel scaffold; baseline (speedup 1.0000x reference)
import jax
import jax.numpy as jnp
from jax.experimental import pallas as pl


def kernel(gold_mention_bounds, gold_mention_bounds_mask, mention_logits, mention_bounds):
    raise NotImplementedError("write your pallas kernel here")


# speedup vs baseline: 3.5100x; 3.5100x over previous
import functools

import jax
import jax.numpy as jnp
from jax import lax
from jax.experimental import pallas as pl
from jax.experimental.pallas import tpu as pltpu

_BS = 16
_NM = 200
_AM = 8192
_KEY_MUL = 16384


def _loss_kernel(g0_ref, g1_ref, gm_ref, c0_ref, c1_ref, x_ref, out_ref, acc_ref):
    b = pl.program_id(0)

    @pl.when(b == 0)
    def _():
        acc_ref[0] = 0.0
        acc_ref[1] = 0.0

    ck = c0_ref[0] * _KEY_MUL + c1_ref[0]

    def body(i, acc):
        m = gm_ref[0, 0, i]
        gk = jnp.where(m != 0, g0_ref[0, 0, i] * _KEY_MUL + g1_ref[0, 0, i] - 1,
                       -_KEY_MUL - 1)
        return acc | (ck == gk).astype(jnp.int32)

    match = lax.fori_loop(0, _NM, body, jnp.zeros((1, _AM), jnp.int32),
                          unroll=8)

    x = x_ref[0]
    y = match.astype(jnp.float32)
    valid = x != -jnp.inf
    bce = jnp.maximum(x, 0.0) - x * y + jnp.log1p(jnp.exp(-jnp.abs(x)))
    acc_ref[0] += jnp.sum(jnp.where(valid, bce, 0.0))
    acc_ref[1] += jnp.sum(valid.astype(jnp.float32))

    @pl.when(b == _BS - 1)
    def _():
        out_ref[0, 0] = acc_ref[0] / acc_ref[1]


@jax.jit
def kernel(gold_mention_bounds, gold_mention_bounds_mask, mention_logits,
           mention_bounds):
    g0 = gold_mention_bounds[:, :, 0].astype(jnp.int32).reshape(_BS, 1, _NM)
    g1 = gold_mention_bounds[:, :, 1].astype(jnp.int32).reshape(_BS, 1, _NM)
    gm = gold_mention_bounds_mask.astype(jnp.int32).reshape(_BS, 1, _NM)
    c0 = mention_bounds[:, :, 0].astype(jnp.int32).reshape(_BS, 1, _AM)
    c1 = mention_bounds[:, :, 1].astype(jnp.int32).reshape(_BS, 1, _AM)
    x = mention_logits.reshape(_BS, 1, _AM)

    smem = functools.partial(pl.BlockSpec, memory_space=pltpu.SMEM)
    out = pl.pallas_call(
        _loss_kernel,
        grid=(_BS,),
        in_specs=[
            smem((1, 1, _NM), lambda b: (b, 0, 0)),
            smem((1, 1, _NM), lambda b: (b, 0, 0)),
            smem((1, 1, _NM), lambda b: (b, 0, 0)),
            pl.BlockSpec((1, 1, _AM), lambda b: (b, 0, 0)),
            pl.BlockSpec((1, 1, _AM), lambda b: (b, 0, 0)),
            pl.BlockSpec((1, 1, _AM), lambda b: (b, 0, 0)),
        ],
        out_specs=smem((1, 1), lambda b: (0, 0)),
        out_shape=jax.ShapeDtypeStruct((1, 1), jnp.float32),
        scratch_shapes=[pltpu.SMEM((2,), jnp.float32)],
    )(g0, g1, gm, c0, c1, x)
    return out.reshape(())
